# Initial kernel scaffold; baseline (speedup 1.0000x reference)
import jax
import jax.numpy as jnp
from jax.experimental import pallas as pl


def kernel(x, batch_ids):
    raise NotImplementedError("write your pallas kernel here")


# speedup vs baseline: 1.1509x; 1.1509x over previous
import jax
import jax.numpy as jnp
from jax import lax
from jax.experimental import pallas as pl
from jax.experimental.pallas import tpu as pltpu

B = 8
NMAX = 512
H = 16
F = 768
N_TOTAL = 2048
NEG = -1000000000.0
CHUNK = 64
WIN = CHUNK + 8


def _kernel(ids_ref, x_ref, dense_ref, mask_ref):
    b = pl.program_id(0)
    h = pl.program_id(1)
    ids = ids_ref[...]
    cnt = jnp.sum((ids == b).astype(jnp.int32))

    @pl.when(h == 0)
    def _():
        cb = jnp.sum((ids < b).astype(jnp.int32))
        a = (cb // 8) * 8
        r = cb - a
        shift = (WIN - r) % WIN
        for j in range(NMAX // CHUNK):
            win = x_ref[pl.ds(a + j * CHUNK, WIN), :]
            rolled = pltpu.roll(win, shift, axis=0)[:CHUNK, :]
            kio = lax.broadcasted_iota(jnp.int32, (CHUNK, 1), 0) + j * CHUNK
            dense_ref[0, pl.ds(j * CHUNK, CHUNK), :] = jnp.where(
                kio < cnt, rolled, 0.0
            )

    col = lax.broadcasted_iota(jnp.int32, (NMAX, NMAX), 1)
    mask_ref[0, 0] = jnp.where(col >= cnt, NEG, 0.0)


def kernel(x, batch_ids):
    x_pad = jnp.concatenate(
        [x, jnp.zeros((NMAX + 8, F), x.dtype)], axis=0
    )
    ids2d = batch_ids.astype(jnp.int32).reshape(16, 128)
    dense_x, attn_mask = pl.pallas_call(
        _kernel,
        grid=(B, H),
        in_specs=[
            pl.BlockSpec((16, 128), lambda b, h: (0, 0)),
            pl.BlockSpec((N_TOTAL + NMAX + 8, F), lambda b, h: (0, 0)),
        ],
        out_specs=[
            pl.BlockSpec((1, NMAX, F), lambda b, h: (b, 0, 0)),
            pl.BlockSpec((1, 1, NMAX, NMAX), lambda b, h: (b, h, 0, 0)),
        ],
        out_shape=[
            jax.ShapeDtypeStruct((B, NMAX, F), x.dtype),
            jax.ShapeDtypeStruct((B, H, NMAX, NMAX), jnp.float32),
        ],
    )(ids2d, x_pad)
    return dense_x, attn_mask


# speedup vs baseline: 1.9887x; 1.7280x over previous
import jax
import jax.numpy as jnp
from jax import lax
from jax.experimental import pallas as pl
from jax.experimental.pallas import tpu as pltpu

B = 8
NMAX = 512
H = 16
F = 768
N_TOTAL = 2048
NEG = -1000000000.0
CHUNK = 64
WIN = CHUNK + 8


def _kernel(ids_ref, x_ref, dense_ref, mask_hbm, tile_ref, sem):
    b = pl.program_id(0)
    ids = ids_ref[...]
    cnt = jnp.sum((ids == b).astype(jnp.int32))
    cb = jnp.sum((ids < b).astype(jnp.int32))

    col = lax.broadcasted_iota(jnp.int32, (NMAX, NMAX), 1)
    tile_ref[...] = jnp.where(col >= cnt, NEG, 0.0)
    for h in range(H):
        pltpu.make_async_copy(tile_ref, mask_hbm.at[b, h], sem).start()

    for j in range(NMAX // CHUNK):
        start = cb + j * CHUNK
        s = jnp.minimum((start // 8) * 8, N_TOTAL - WIN)
        d = start - s
        win = x_ref[pl.ds(s, WIN), :]
        rolled = pltpu.roll(win, (WIN - d) % WIN, axis=0)[:CHUNK, :]
        kio = lax.broadcasted_iota(jnp.int32, (CHUNK, 1), 0) + j * CHUNK
        dense_ref[0, pl.ds(j * CHUNK, CHUNK), :] = jnp.where(
            kio < cnt, rolled, 0.0
        )

    for h in range(H):
        pltpu.make_async_copy(tile_ref, mask_hbm.at[b, h], sem).wait()


def kernel(x, batch_ids):
    ids2d = batch_ids.astype(jnp.int32).reshape(16, 128)
    dense_x, attn_mask = pl.pallas_call(
        _kernel,
        grid=(B,),
        in_specs=[
            pl.BlockSpec((16, 128), lambda b: (0, 0)),
            pl.BlockSpec((N_TOTAL, F), lambda b: (0, 0)),
        ],
        out_specs=[
            pl.BlockSpec((1, NMAX, F), lambda b: (b, 0, 0)),
            pl.BlockSpec(memory_space=pl.ANY),
        ],
        out_shape=[
            jax.ShapeDtypeStruct((B, NMAX, F), x.dtype),
            jax.ShapeDtypeStruct((B, H, NMAX, NMAX), jnp.float32),
        ],
        scratch_shapes=[
            pltpu.VMEM((NMAX, NMAX), jnp.float32),
            pltpu.SemaphoreType.DMA,
        ],
    )(ids2d, x)
    return dense_x, attn_mask


# speedup vs baseline: 2.2069x; 1.1097x over previous
import jax
import jax.numpy as jnp
from jax import lax
from jax.experimental import pallas as pl
from jax.experimental.pallas import tpu as pltpu

B = 8
NMAX = 512
H = 16
F = 768
N_TOTAL = 2048
NEG = -1000000000.0
CHUNK = 64
WIN = CHUNK + 8


def _kernel(ids_ref, x_ref, dense_ref, mask_hbm, tile_ref, sem):
    b = pl.program_id(0)
    p = lax.rem(b, 2)
    ids = ids_ref[...]
    cnt = jnp.sum((ids == b).astype(jnp.int32))
    cb = jnp.sum((ids < b).astype(jnp.int32))

    @pl.when(b >= 2)
    def _():
        for h in range(H):
            pltpu.make_async_copy(
                tile_ref.at[p], mask_hbm.at[b - 2, h], sem.at[p]
            ).wait()

    col = lax.broadcasted_iota(jnp.int32, (NMAX, NMAX), 1)
    tile_ref[p] = jnp.where(col >= cnt, NEG, 0.0)
    for h in range(H):
        pltpu.make_async_copy(tile_ref.at[p], mask_hbm.at[b, h], sem.at[p]).start()

    for j in range(NMAX // CHUNK):
        start = cb + j * CHUNK
        s = jnp.minimum((start // 8) * 8, N_TOTAL - WIN)
        d = start - s
        win = x_ref[pl.ds(s, WIN), :]
        rolled = pltpu.roll(win, (WIN - d) % WIN, axis=0)[:CHUNK, :]
        kio = lax.broadcasted_iota(jnp.int32, (CHUNK, 1), 0) + j * CHUNK
        dense_ref[0, pl.ds(j * CHUNK, CHUNK), :] = jnp.where(
            kio < cnt, rolled, 0.0
        )

    @pl.when(b == B - 1)
    def _():
        for h in range(H):
            pltpu.make_async_copy(
                tile_ref.at[1 - p], mask_hbm.at[b - 1, h], sem.at[1 - p]
            ).wait()
        for h in range(H):
            pltpu.make_async_copy(
                tile_ref.at[p], mask_hbm.at[b, h], sem.at[p]
            ).wait()


def kernel(x, batch_ids):
    ids2d = batch_ids.astype(jnp.int32).reshape(16, 128)
    dense_x, attn_mask = pl.pallas_call(
        _kernel,
        grid=(B,),
        in_specs=[
            pl.BlockSpec((16, 128), lambda b: (0, 0)),
            pl.BlockSpec((N_TOTAL, F), lambda b: (0, 0)),
        ],
        out_specs=[
            pl.BlockSpec((1, NMAX, F), lambda b: (b, 0, 0)),
            pl.BlockSpec(memory_space=pl.ANY),
        ],
        out_shape=[
            jax.ShapeDtypeStruct((B, NMAX, F), x.dtype),
            jax.ShapeDtypeStruct((B, H, NMAX, NMAX), jnp.float32),
        ],
        scratch_shapes=[
            pltpu.VMEM((2, NMAX, NMAX), jnp.float32),
            pltpu.SemaphoreType.DMA((2,)),
        ],
    )(ids2d, x)
    return dense_x, attn_mask


# speedup vs baseline: 2.3355x; 1.0583x over previous
import jax
import jax.numpy as jnp
from jax import lax
from jax.experimental import pallas as pl
from jax.experimental.pallas import tpu as pltpu

B = 8
NMAX = 512
H = 16
F = 768
N_TOTAL = 2048
NEG = -1000000000.0
CHUNK = 64
WIN = CHUNK + 8


def _kernel(ids_ref, x_hbm, dense_hbm, mask_hbm, xv, tiles, dsc,
            semx, semm, semd):
    pltpu.make_async_copy(x_hbm, xv, semx).start()

    ids = ids_ref[...]
    cnts = [jnp.sum((ids == b).astype(jnp.int32)) for b in range(B)]
    cbs = [jnp.sum((ids < b).astype(jnp.int32)) for b in range(B)]

    col = lax.broadcasted_iota(jnp.int32, (NMAX, NMAX), 1)
    for b in range(B):
        tiles[b] = jnp.where(col >= cnts[b], NEG, 0.0)
        for h in range(H):
            pltpu.make_async_copy(tiles.at[b], mask_hbm.at[b, h], semm).start()

    pltpu.make_async_copy(x_hbm, xv, semx).wait()
    kio = lax.broadcasted_iota(jnp.int32, (CHUNK, 1), 0)
    for b in range(B):
        for j in range(NMAX // CHUNK):
            start = cbs[b] + j * CHUNK
            s = jnp.minimum((start // 8) * 8, N_TOTAL - WIN)
            d = start - s
            win = xv[pl.ds(s, WIN), :]
            rolled = pltpu.roll(win, (WIN - d) % WIN, axis=0)[:CHUNK, :]
            dsc[b, pl.ds(j * CHUNK, CHUNK), :] = jnp.where(
                kio + j * CHUNK < cnts[b], rolled, 0.0
            )
        pltpu.make_async_copy(dsc.at[b], dense_hbm.at[b], semd).start()

    for b in range(B):
        pltpu.make_async_copy(dsc.at[b], dense_hbm.at[b], semd).wait()
    for b in range(B):
        for h in range(H):
            pltpu.make_async_copy(tiles.at[b], mask_hbm.at[b, h], semm).wait()


def kernel(x, batch_ids):
    ids2d = batch_ids.astype(jnp.int32).reshape(16, 128)
    dense_x, attn_mask = pl.pallas_call(
        _kernel,
        in_specs=[
            pl.BlockSpec((16, 128), lambda: (0, 0)),
            pl.BlockSpec(memory_space=pl.ANY),
        ],
        out_specs=[
            pl.BlockSpec(memory_space=pl.ANY),
            pl.BlockSpec(memory_space=pl.ANY),
        ],
        out_shape=[
            jax.ShapeDtypeStruct((B, NMAX, F), x.dtype),
            jax.ShapeDtypeStruct((B, H, NMAX, NMAX), jnp.float32),
        ],
        scratch_shapes=[
            pltpu.VMEM((N_TOTAL, F), jnp.float32),
            pltpu.VMEM((B, NMAX, NMAX), jnp.float32),
            pltpu.VMEM((B, NMAX, F), jnp.float32),
            pltpu.SemaphoreType.DMA,
            pltpu.SemaphoreType.DMA,
            pltpu.SemaphoreType.DMA,
        ],
    )(ids2d, x)
    return dense_x, attn_mask
